# Initial kernel scaffold; baseline (speedup 1.0000x reference)
import jax
import jax.numpy as jnp
from jax.experimental import pallas as pl


def kernel(nfeats, efeats, edge_index, W_apply, b_apply, W_edge, b_edge):
    raise NotImplementedError("write your pallas kernel here")


# speedup vs baseline: 2.4298x; 2.4298x over previous
import functools

import jax
import jax.numpy as jnp
from jax import lax
from jax.experimental import pallas as pl
from jax.experimental.pallas import tpu as pltpu
from jax.experimental.pallas import tpu_sc as plsc

N = 10000
E = 320000
DIN = 128
DE = 16
DOUT = 128
DEDGE = 256

NC = 2
NS = 16
NW = NC * NS

CHUNK = 128
NCHUNK = E // CHUNK
BASE_K = NCHUNK // NW
EXTRA_W = NCHUNK - BASE_K * NW

STRIPE = 624
LAST_STRIPE = N - 15 * STRIPE

_mesh = plsc.VectorSubcoreMesh(core_axis_name="c", subcore_axis_name="s")


def _worker_id():
    return lax.axis_index("s") * NC + lax.axis_index("c")


def _num_chunks(wid):
    return BASE_K + jnp.where(wid < EXTRA_W, 1, 0)


WPT = N * DE // NS


@functools.partial(
    pl.kernel,
    out_type=(
        jax.ShapeDtypeStruct((N * DE,), jnp.float32),
        jax.ShapeDtypeStruct((N * DE,), jnp.float32),
        jax.ShapeDtypeStruct((N,), jnp.float32),
        jax.ShapeDtypeStruct((N,), jnp.float32),
    ),
    mesh=_mesh,
    scratch_types=[
        pltpu.VMEM_SHARED((N * DE,), jnp.float32),
        pltpu.VMEM_SHARED((N,), jnp.float32),
        pltpu.VMEM((DE + 1, CHUNK), jnp.float32),
        pltpu.VMEM((DE + 1, CHUNK), jnp.int32),
        pltpu.VMEM((WPT,), jnp.float32),
    ],
)
def _sc_segment(eft_hbm, idx_hbm, msum0_out, msum1_out, cnt0_out, cnt1_out,
                msum_sp, cnt_sp, colbuf, idxbuf, zflat):
    cid = lax.axis_index("c")
    sid = lax.axis_index("s")
    wid = _worker_id()

    zeros16 = jnp.zeros((16,), jnp.float32)

    def zrow(r, _):
        zflat[pl.ds(r * 16, 16)] = zeros16
        return _

    lax.fori_loop(0, WPT // 16, zrow, 0)

    pltpu.sync_copy(zflat, msum_sp.at[pl.ds(sid * WPT, WPT)])

    @pl.when(sid < 15)
    def _():
        pltpu.sync_copy(zflat.at[pl.ds(0, STRIPE)],
                        cnt_sp.at[pl.ds(sid * STRIPE, STRIPE)])

    @pl.when(sid == 15)
    def _():
        pltpu.sync_copy(zflat.at[pl.ds(0, LAST_STRIPE)],
                        cnt_sp.at[pl.ds(15 * STRIPE, LAST_STRIPE)])

    plsc.subcore_barrier()

    def body(k, _):
        c = wid + k * NW
        base = c * CHUNK
        pltpu.sync_copy(idx_hbm.at[:, pl.ds(base, CHUNK)], idxbuf)
        pltpu.sync_copy(eft_hbm.at[:, pl.ds(base, CHUNK)], colbuf)
        for col in range(DE):
            pltpu.sync_copy(colbuf.at[col], msum_sp.at[idxbuf.at[col]], add=True)
        pltpu.sync_copy(colbuf.at[DE], cnt_sp.at[idxbuf.at[DE]], add=True)
        return _

    lax.fori_loop(0, _num_chunks(wid), body, 0)

    plsc.subcore_barrier()

    pltpu.sync_copy(msum_sp.at[pl.ds(sid * WPT, WPT)], zflat)

    @pl.when(cid == 0)
    def _():
        pltpu.sync_copy(zflat, msum0_out.at[pl.ds(sid * WPT, WPT)])

    @pl.when(cid == 1)
    def _():
        pltpu.sync_copy(zflat, msum1_out.at[pl.ds(sid * WPT, WPT)])

    @pl.when(sid < 15)
    def _():
        pltpu.sync_copy(cnt_sp.at[pl.ds(sid * STRIPE, STRIPE)],
                        zflat.at[pl.ds(0, STRIPE)])

    @pl.when(sid == 15)
    def _():
        pltpu.sync_copy(cnt_sp.at[pl.ds(15 * STRIPE, LAST_STRIPE)],
                        zflat.at[pl.ds(0, LAST_STRIPE)])

    @pl.when(jnp.logical_and(sid < 15, cid == 0))
    def _():
        pltpu.sync_copy(zflat.at[pl.ds(0, STRIPE)],
                        cnt0_out.at[pl.ds(sid * STRIPE, STRIPE)])

    @pl.when(jnp.logical_and(sid == 15, cid == 0))
    def _():
        pltpu.sync_copy(zflat.at[pl.ds(0, LAST_STRIPE)],
                        cnt0_out.at[pl.ds(15 * STRIPE, LAST_STRIPE)])

    @pl.when(jnp.logical_and(sid < 15, cid == 1))
    def _():
        pltpu.sync_copy(zflat.at[pl.ds(0, STRIPE)],
                        cnt1_out.at[pl.ds(sid * STRIPE, STRIPE)])

    @pl.when(jnp.logical_and(sid == 15, cid == 1))
    def _():
        pltpu.sync_copy(zflat.at[pl.ds(0, LAST_STRIPE)],
                        cnt1_out.at[pl.ds(15 * STRIPE, LAST_STRIPE)])


NB = 10
BN = N // NB


def _tc_body(nf_ref, m0_ref, m1_ref, c0_ref, c1_ref, wa_ref, ba_ref, we_ref,
             be_ref, h_ref, hpu_ref, hpv_ref):
    msum = m0_ref[...] + m1_ref[...]
    cnt = c0_ref[0, 0, :] + c1_ref[0, 0, :]
    recip = 1.0 / jnp.maximum(cnt, 1.0)
    h_neigh = msum * recip[:, None]
    x = jnp.dot(nf_ref[...], wa_ref[0:DIN, :],
                preferred_element_type=jnp.float32)
    x += jnp.dot(h_neigh, wa_ref[DIN:DIN + DE, :],
                 preferred_element_type=jnp.float32)
    h = jnp.maximum(x + ba_ref[...], 0.0)
    h_ref[...] = h
    hpu_ref[...] = jnp.dot(h, we_ref[0:DOUT, :],
                           preferred_element_type=jnp.float32) + be_ref[...]
    hpv_ref[...] = jnp.dot(h, we_ref[DOUT:2 * DOUT, :],
                           preferred_element_type=jnp.float32)


def _tc_dense(nf, msum0, msum1, cnt0, cnt1, W_apply, b_apply, W_edge, b_edge):
    return pl.pallas_call(
        _tc_body,
        grid=(NB,),
        in_specs=[
            pl.BlockSpec((BN, DIN), lambda i: (i, 0)),
            pl.BlockSpec((BN, DE), lambda i: (i, 0)),
            pl.BlockSpec((BN, DE), lambda i: (i, 0)),
            pl.BlockSpec((1, 1, BN), lambda i: (i, 0, 0)),
            pl.BlockSpec((1, 1, BN), lambda i: (i, 0, 0)),
            pl.BlockSpec((DIN + DE, DOUT), lambda i: (0, 0)),
            pl.BlockSpec((1, DOUT), lambda i: (0, 0)),
            pl.BlockSpec((2 * DOUT, DEDGE), lambda i: (0, 0)),
            pl.BlockSpec((1, DEDGE), lambda i: (0, 0)),
        ],
        out_specs=[
            pl.BlockSpec((BN, DOUT), lambda i: (i, 0)),
            pl.BlockSpec((BN, DEDGE), lambda i: (i, 0)),
            pl.BlockSpec((BN, DEDGE), lambda i: (i, 0)),
        ],
        out_shape=[
            jax.ShapeDtypeStruct((N, DOUT), jnp.float32),
            jax.ShapeDtypeStruct((N, DEDGE), jnp.float32),
            jax.ShapeDtypeStruct((N, DEDGE), jnp.float32),
        ],
    )(nf, msum0.reshape(N, DE), msum1.reshape(N, DE),
      cnt0.reshape(NB, 1, BN), cnt1.reshape(NB, 1, BN),
      W_apply, b_apply, W_edge, b_edge)


@functools.partial(
    pl.kernel,
    out_type=jax.ShapeDtypeStruct((E, DEDGE), jnp.float32),
    mesh=_mesh,
    scratch_types=[
        pltpu.VMEM((1, CHUNK), jnp.int32),
        pltpu.VMEM((1, CHUNK), jnp.int32),
        pltpu.VMEM((CHUNK, DEDGE), jnp.float32),
        pltpu.VMEM((CHUNK, DEDGE), jnp.float32),
    ],
)
def _sc_edge(hpu_hbm, hpv_hbm, u_hbm, v_hbm, out_hbm, idxu, idxv, bufu, bufv):
    wid = _worker_id()

    def body(k, _):
        c = wid + k * NW
        base = c * CHUNK
        pltpu.sync_copy(u_hbm.at[pl.ds(base, CHUNK)], idxu.at[0])
        pltpu.sync_copy(v_hbm.at[pl.ds(base, CHUNK)], idxv.at[0])
        pltpu.sync_copy(hpu_hbm.at[idxu.at[0]], bufu)
        pltpu.sync_copy(hpv_hbm.at[idxv.at[0]], bufv)

        def add_row(r, _):
            for j in range(DEDGE // 16):
                x = bufv[r, pl.ds(j * 16, 16)]
                plsc.addupdate(bufu.at[r, pl.ds(j * 16, 16)], x)
            return _

        lax.fori_loop(0, CHUNK, add_row, 0)
        pltpu.sync_copy(bufu, out_hbm.at[pl.ds(base, CHUNK)])
        return _

    lax.fori_loop(0, _num_chunks(wid), body, 0)


def kernel(nfeats, efeats, edge_index, W_apply, b_apply, W_edge, b_edge):
    nf = nfeats.reshape(N, DIN)
    ef = efeats.reshape(E, DE)
    ei = edge_index.astype(jnp.int32)
    u = ei[0]
    v = ei[1]
    eft = jnp.concatenate([ef.T, jnp.ones((1, E), jnp.float32)], axis=0)
    idx_all = jnp.concatenate(
        [v[None, :] * DE + jnp.arange(DE, dtype=jnp.int32)[:, None],
         v[None, :]], axis=0)
    msum0, msum1, cnt0, cnt1 = _sc_segment(eft, idx_all)
    h, hpu, hpv = _tc_dense(nf, msum0, msum1, cnt0, cnt1, W_apply,
                            b_apply.reshape(1, DOUT), W_edge,
                            b_edge.reshape(1, DEDGE))
    edge = _sc_edge(hpu, hpv, u, v)
    return h.reshape(N, 1, DOUT), edge.reshape(E, 1, DEDGE)


# speedup vs baseline: 3.1092x; 1.2796x over previous
import functools

import jax
import jax.numpy as jnp
from jax import lax
from jax.experimental import pallas as pl
from jax.experimental.pallas import tpu as pltpu
from jax.experimental.pallas import tpu_sc as plsc

N = 10000
E = 320000
DIN = 128
DE = 16
DOUT = 128
DEDGE = 256

NC = 2
NS = 16
NW = NC * NS

CHUNK = 128
NCHUNK = E // CHUNK
BASE_K = NCHUNK // NW
EXTRA_W = NCHUNK - BASE_K * NW

STRIPE = 624
LAST_STRIPE = N - 15 * STRIPE

_mesh = plsc.VectorSubcoreMesh(core_axis_name="c", subcore_axis_name="s")


def _worker_id():
    return lax.axis_index("s") * NC + lax.axis_index("c")


def _num_chunks(wid):
    return BASE_K + jnp.where(wid < EXTRA_W, 1, 0)


WPT = N * DE // NS


@functools.partial(
    pl.kernel,
    out_type=(
        jax.ShapeDtypeStruct((N * DE,), jnp.float32),
        jax.ShapeDtypeStruct((N * DE,), jnp.float32),
        jax.ShapeDtypeStruct((N,), jnp.float32),
        jax.ShapeDtypeStruct((N,), jnp.float32),
    ),
    mesh=_mesh,
    scratch_types=[
        pltpu.VMEM_SHARED((N * DE,), jnp.float32),
        pltpu.VMEM_SHARED((N,), jnp.float32),
        pltpu.VMEM((DE + 1, CHUNK), jnp.float32),
        pltpu.VMEM((DE + 1, CHUNK), jnp.int32),
        pltpu.VMEM((WPT,), jnp.float32),
    ],
)
def _sc_segment(eft_hbm, idx_hbm, msum0_out, msum1_out, cnt0_out, cnt1_out,
                msum_sp, cnt_sp, colbuf, idxbuf, zflat):
    cid = lax.axis_index("c")
    sid = lax.axis_index("s")
    wid = _worker_id()

    zeros16 = jnp.zeros((16,), jnp.float32)

    def zrow(r, _):
        zflat[pl.ds(r * 16, 16)] = zeros16
        return _

    lax.fori_loop(0, WPT // 16, zrow, 0)

    pltpu.sync_copy(zflat, msum_sp.at[pl.ds(sid * WPT, WPT)])

    @pl.when(sid < 15)
    def _():
        pltpu.sync_copy(zflat.at[pl.ds(0, STRIPE)],
                        cnt_sp.at[pl.ds(sid * STRIPE, STRIPE)])

    @pl.when(sid == 15)
    def _():
        pltpu.sync_copy(zflat.at[pl.ds(0, LAST_STRIPE)],
                        cnt_sp.at[pl.ds(15 * STRIPE, LAST_STRIPE)])

    plsc.subcore_barrier()

    def body(k, _):
        c = wid + k * NW
        base = c * CHUNK
        pltpu.sync_copy(idx_hbm.at[:, pl.ds(base, CHUNK)], idxbuf)
        pltpu.sync_copy(eft_hbm.at[:, pl.ds(base, CHUNK)], colbuf)
        for col in range(DE):
            pltpu.sync_copy(colbuf.at[col], msum_sp.at[idxbuf.at[col]], add=True)
        pltpu.sync_copy(colbuf.at[DE], cnt_sp.at[idxbuf.at[DE]], add=True)
        return _

    lax.fori_loop(0, _num_chunks(wid), body, 0)

    plsc.subcore_barrier()

    pltpu.sync_copy(msum_sp.at[pl.ds(sid * WPT, WPT)], zflat)

    @pl.when(cid == 0)
    def _():
        pltpu.sync_copy(zflat, msum0_out.at[pl.ds(sid * WPT, WPT)])

    @pl.when(cid == 1)
    def _():
        pltpu.sync_copy(zflat, msum1_out.at[pl.ds(sid * WPT, WPT)])

    @pl.when(sid < 15)
    def _():
        pltpu.sync_copy(cnt_sp.at[pl.ds(sid * STRIPE, STRIPE)],
                        zflat.at[pl.ds(0, STRIPE)])

    @pl.when(sid == 15)
    def _():
        pltpu.sync_copy(cnt_sp.at[pl.ds(15 * STRIPE, LAST_STRIPE)],
                        zflat.at[pl.ds(0, LAST_STRIPE)])

    @pl.when(jnp.logical_and(sid < 15, cid == 0))
    def _():
        pltpu.sync_copy(zflat.at[pl.ds(0, STRIPE)],
                        cnt0_out.at[pl.ds(sid * STRIPE, STRIPE)])

    @pl.when(jnp.logical_and(sid == 15, cid == 0))
    def _():
        pltpu.sync_copy(zflat.at[pl.ds(0, LAST_STRIPE)],
                        cnt0_out.at[pl.ds(15 * STRIPE, LAST_STRIPE)])

    @pl.when(jnp.logical_and(sid < 15, cid == 1))
    def _():
        pltpu.sync_copy(zflat.at[pl.ds(0, STRIPE)],
                        cnt1_out.at[pl.ds(sid * STRIPE, STRIPE)])

    @pl.when(jnp.logical_and(sid == 15, cid == 1))
    def _():
        pltpu.sync_copy(zflat.at[pl.ds(0, LAST_STRIPE)],
                        cnt1_out.at[pl.ds(15 * STRIPE, LAST_STRIPE)])


NB = 10
BN = N // NB


def _tc_body(nf_ref, m0_ref, m1_ref, c0_ref, c1_ref, wa_ref, ba_ref, we_ref,
             be_ref, h_ref, hpu_ref, hpv_ref):
    msum = m0_ref[...] + m1_ref[...]
    cnt = c0_ref[0, 0, :] + c1_ref[0, 0, :]
    recip = 1.0 / jnp.maximum(cnt, 1.0)
    h_neigh = msum * recip[:, None]
    x = jnp.dot(nf_ref[...], wa_ref[0:DIN, :],
                preferred_element_type=jnp.float32)
    x += jnp.dot(h_neigh, wa_ref[DIN:DIN + DE, :],
                 preferred_element_type=jnp.float32)
    h = jnp.maximum(x + ba_ref[...], 0.0)
    h_ref[...] = h
    hpu_ref[...] = jnp.dot(h, we_ref[0:DOUT, :],
                           preferred_element_type=jnp.float32) + be_ref[...]
    hpv_ref[...] = jnp.dot(h, we_ref[DOUT:2 * DOUT, :],
                           preferred_element_type=jnp.float32)


def _tc_dense(nf, msum0, msum1, cnt0, cnt1, W_apply, b_apply, W_edge, b_edge):
    return pl.pallas_call(
        _tc_body,
        grid=(NB,),
        in_specs=[
            pl.BlockSpec((BN, DIN), lambda i: (i, 0)),
            pl.BlockSpec((BN, DE), lambda i: (i, 0)),
            pl.BlockSpec((BN, DE), lambda i: (i, 0)),
            pl.BlockSpec((1, 1, BN), lambda i: (i, 0, 0)),
            pl.BlockSpec((1, 1, BN), lambda i: (i, 0, 0)),
            pl.BlockSpec((DIN + DE, DOUT), lambda i: (0, 0)),
            pl.BlockSpec((1, DOUT), lambda i: (0, 0)),
            pl.BlockSpec((2 * DOUT, DEDGE), lambda i: (0, 0)),
            pl.BlockSpec((1, DEDGE), lambda i: (0, 0)),
        ],
        out_specs=[
            pl.BlockSpec((BN, DOUT), lambda i: (i, 0)),
            pl.BlockSpec((BN, DEDGE), lambda i: (i, 0)),
            pl.BlockSpec((BN, DEDGE), lambda i: (i, 0)),
        ],
        out_shape=[
            jax.ShapeDtypeStruct((N, DOUT), jnp.float32),
            jax.ShapeDtypeStruct((N, DEDGE), jnp.float32),
            jax.ShapeDtypeStruct((N, DEDGE), jnp.float32),
        ],
    )(nf, msum0.reshape(N, DE), msum1.reshape(N, DE),
      cnt0.reshape(NB, 1, BN), cnt1.reshape(NB, 1, BN),
      W_apply, b_apply, W_edge, b_edge)


DHALF = DEDGE // 2


@functools.partial(
    pl.kernel,
    out_type=jax.ShapeDtypeStruct((E, DEDGE), jnp.float32),
    mesh=_mesh,
    scratch_types=[
        pltpu.VMEM((2, CHUNK), jnp.int32),
        pltpu.VMEM((2, CHUNK), jnp.int32),
        pltpu.VMEM((CHUNK, DHALF), jnp.float32),
        pltpu.VMEM((CHUNK, DHALF), jnp.float32),
        pltpu.VMEM((CHUNK, DHALF), jnp.float32),
        pltpu.VMEM((CHUNK, DHALF), jnp.float32),
        pltpu.SemaphoreType.DMA,
        pltpu.SemaphoreType.DMA,
        pltpu.SemaphoreType.DMA,
        pltpu.SemaphoreType.DMA,
    ],
)
def _sc_edge(tuA, tvA, tuB, tvB, ei_hbm, out_hbm,
             idxb0, idxb1, bu0, bv0, bu1, bv1, su0, sv0, su1, sv1):
    wid = _worker_id()
    nk = _num_chunks(wid)
    slots = ((idxb0, bu0, bv0, su0, sv0), (idxb1, bu1, bv1, su1, sv1))

    for tu, tv, coff in ((tuA, tvA, 0), (tuB, tvB, DHALF)):

        def load_fire(k, slot):
            idxb, bu, bv, su, sv = slot
            base = (wid + k * NW) * CHUNK
            pltpu.sync_copy(ei_hbm.at[:, pl.ds(base, CHUNK)], idxb)
            pltpu.async_copy(tu.at[idxb.at[0]], bu, su)
            pltpu.async_copy(tv.at[idxb.at[1]], bv, sv)

        @pl.when(nk > 0)
        def _():
            load_fire(0, slots[0])

        def pair(p, carry):
            for b in (0, 1):
                k = 2 * p + b
                idxb, bu, bv, su, sv = slots[b]

                @pl.when(k < nk)
                def _():
                    @pl.when(k + 1 < nk)
                    def _():
                        load_fire(k + 1, slots[1 - b])

                    pltpu.make_async_copy(tu.at[idxb.at[0]], bu, su).wait()
                    pltpu.make_async_copy(tv.at[idxb.at[1]], bv, sv).wait()

                    def add_row(r, _2):
                        for j in range(DHALF // 16):
                            x = bv[r, pl.ds(j * 16, 16)]
                            plsc.addupdate(bu.at[r, pl.ds(j * 16, 16)], x)
                        return _2

                    lax.fori_loop(0, CHUNK, add_row, 0)
                    base = (wid + k * NW) * CHUNK
                    pltpu.sync_copy(
                        bu, out_hbm.at[pl.ds(base, CHUNK), pl.ds(coff, DHALF)])
            return carry

        lax.fori_loop(0, (nk + 1) // 2, pair, 0)


def kernel(nfeats, efeats, edge_index, W_apply, b_apply, W_edge, b_edge):
    nf = nfeats.reshape(N, DIN)
    ef = efeats.reshape(E, DE)
    ei = edge_index.astype(jnp.int32)
    u = ei[0]
    v = ei[1]
    eft = jnp.concatenate([ef.T, jnp.ones((1, E), jnp.float32)], axis=0)
    idx_all = jnp.concatenate(
        [v[None, :] * DE + jnp.arange(DE, dtype=jnp.int32)[:, None],
         v[None, :]], axis=0)
    msum0, msum1, cnt0, cnt1 = _sc_segment(eft, idx_all)
    h, hpu, hpv = _tc_dense(nf, msum0, msum1, cnt0, cnt1, W_apply,
                            b_apply.reshape(1, DOUT), W_edge,
                            b_edge.reshape(1, DEDGE))
    edge = _sc_edge(hpu[:, :DHALF], hpv[:, :DHALF],
                    hpu[:, DHALF:], hpv[:, DHALF:], ei)
    return h.reshape(N, 1, DOUT), edge.reshape(E, 1, DEDGE)
